# baseline (speedup 1.0000x reference)
import jax
import jax.numpy as jnp
from jax import lax
from jax.experimental import pallas as pl
from jax.experimental.pallas import tpu as pltpu
from jax.experimental.pallas import tpu_sc as plsc

_BATCH = 16384
_D = 3200
_NUM_GROUPS = 10
_NUM_SLICES = 5
_CHUNK = 32
_GROUP_W = _NUM_SLICES * _CHUNK
_USED_D = _NUM_GROUPS * _NUM_SLICES * _CHUNK

_info = plsc.get_sparse_core_info()
_NC = _info.num_cores
_NS = _info.num_subcores
_NW = _NC * _NS
_RPW = _BATCH // _NW
_CB = 128
_NBB = _RPW // _CB
_NCT = (_USED_D + 127) // 128


def _body(in_hbm, iota_hbm, *rest):
    outs = rest[:_NUM_GROUPS]
    in_bufs = rest[_NUM_GROUPS : _NUM_GROUPS + 2]
    st_bufs = rest[_NUM_GROUPS + 2 : _NUM_GROUPS + 4]
    iota_v = rest[_NUM_GROUPS + 4]
    rsems = rest[_NUM_GROUPS + 5 : _NUM_GROUPS + 7]
    wsems = rest[_NUM_GROUPS + 7 : _NUM_GROUPS + 9]
    wid = lax.axis_index("s") * _NC + lax.axis_index("c")
    row0 = wid * _RPW

    pltpu.make_async_copy(iota_hbm, iota_v, rsems[0]).start()
    pltpu.make_async_copy(iota_hbm, iota_v, rsems[0]).wait()
    crows16 = [iota_v[pl.ds(0, 16)] + (16 * k) for k in range(8)]

    def adv(bb, ct):
        wrap = ct + 1 == _NCT
        return (
            jnp.where(wrap, bb + 1, bb),
            jnp.where(wrap, 0, ct + 1),
        )

    def read_desc(bb, ct, u):
        return pltpu.make_async_copy(
            in_hbm.at[pl.ds(row0 + bb * _CB, _CB), pl.ds(ct * 128, 128)],
            in_bufs[u],
            rsems[u],
        )

    def write_desc(bb, ct, u, q):
        m = ct * 4 + q
        g = m % _NUM_GROUPS
        j = m // _NUM_GROUPS
        return pltpu.make_async_copy(
            st_bufs[u].at[pl.ds(q * _CHUNK, _CHUNK), pl.ds(0, _CB)],
            outs[g].at[
                pl.ds(j * _CHUNK, _CHUNK),
                pl.ds(row0 + bb * _CB, _CB),
            ],
            wsems[u],
        )

    def start_writes(bb, ct, u):
        for cts in range(_NCT):
            nq = 2 if cts == _NCT - 1 else 4

            @pl.when(ct == cts)
            def _():
                for q in range(nq):
                    write_desc(bb, cts, u, q).start()

    def wait_writes(bb, ct, u):
        for cts in range(_NCT):
            nq = 2 if cts == _NCT - 1 else 4

            @pl.when(ct == cts)
            def _():
                for q in range(nq):
                    write_desc(bb, cts, u, q).wait()

    nphase = _NBB * _NCT
    read_desc(0, 0, 0).start()
    read_desc(0, 1, 1).start()

    def body(p2, carry):
        bb, ct, pb0, pc0, pb1, pc1 = carry
        cs = [(bb, ct)]
        for _ in range(3):
            cs.append(adv(*cs[-1]))
        pend = [(pb0, pc0), (pb1, pc1)]
        new_pend = []
        for u in (0, 1):
            cbb, cct = cs[u]
            read_desc(cbb, cct, u).wait()

            @pl.when(p2 >= 1)
            def _():
                wait_writes(pend[u][0], pend[u][1], u)

            @plsc.parallel_loop(0, _CB, step=1, unroll=2)
            def _(b):
                colb = jnp.zeros((16,), jnp.int32) + b
                for k in range(8):
                    plsc.store_scatter(
                        st_bufs[u],
                        [crows16[k], colb],
                        in_bufs[u][b, pl.ds(k * 16, 16)],
                    )

            start_writes(cbb, cct, u)
            nbb, nct = cs[u + 2]

            @pl.when(p2 * 2 + u + 2 < nphase)
            def _():
                read_desc(nbb, nct, u).start()

            new_pend.append(cs[u])
        return (
            cs[2][0],
            cs[2][1],
            new_pend[0][0],
            new_pend[0][1],
            new_pend[1][0],
            new_pend[1][1],
        )

    z = jnp.int32(0)
    fin = lax.fori_loop(0, nphase // 2, body, (z, z, z, z, z, z))
    wait_writes(fin[2], fin[3], 0)
    wait_writes(fin[4], fin[5], 1)


def kernel(input_tensor):
    iota = jnp.arange(16, dtype=jnp.int32)
    out_type = [
        jax.ShapeDtypeStruct((_GROUP_W, _BATCH), jnp.float32)
    ] * _NUM_GROUPS
    f = pl.kernel(
        _body,
        out_type=out_type,
        mesh=plsc.VectorSubcoreMesh(core_axis_name="c", subcore_axis_name="s"),
        scratch_types=(
            [pltpu.VMEM((_CB, 128), jnp.float32)] * 2
            + [pltpu.VMEM((128, 129), jnp.float32)] * 2
            + [pltpu.VMEM((16,), jnp.int32)]
            + [pltpu.SemaphoreType.DMA] * 4
        ),
        compiler_params=pltpu.CompilerParams(
            use_tc_tiling_on_sc=True, needs_layout_passes=False
        ),
    )
    outs = f(input_tensor, iota)
    return tuple(jnp.transpose(o) for o in outs)


# speedup vs baseline: 1.5066x; 1.5066x over previous
import jax
import jax.numpy as jnp
from jax import lax
from jax.experimental import pallas as pl
from jax.experimental.pallas import tpu as pltpu
from jax.experimental.pallas import tpu_sc as plsc

_BATCH = 16384
_D = 3200
_NUM_GROUPS = 10
_NUM_SLICES = 5
_CHUNK = 32
_GROUP_W = _NUM_SLICES * _CHUNK
_READ_W = 1664

_info = plsc.get_sparse_core_info()
_NC = _info.num_cores
_NS = _info.num_subcores
_NW = _NC * _NS
_RPW = _BATCH // _NW
_CR = 8
_NCHUNK = _RPW // _CR


def _body(in_hbm, *rest):
    outs = rest[:_NUM_GROUPS]
    in_bufs = rest[_NUM_GROUPS : _NUM_GROUPS + 2]
    out_bufs = [
        rest[_NUM_GROUPS + 2 + u * _NUM_GROUPS :][:_NUM_GROUPS]
        for u in (0, 1)
    ]
    sems = rest[_NUM_GROUPS + 2 + 2 * _NUM_GROUPS :]
    rsems = sems[0:2]
    wsems = sems[2:4]
    wid = lax.axis_index("s") * _NC + lax.axis_index("c")
    row0 = wid * _RPW

    def read_desc(c, u):
        return pltpu.make_async_copy(
            in_hbm.at[pl.ds(row0 + c * _CR, _CR), pl.ds(0, _READ_W)],
            in_bufs[u],
            rsems[u],
        )

    def write_desc(c, u, g):
        return pltpu.make_async_copy(
            out_bufs[u][g],
            outs[g].at[pl.ds(row0 + c * _CR, _CR), :],
            wsems[u],
        )

    read_desc(0, 0).start()
    read_desc(1, 1).start()

    def chunk_pair(c2, _):
        for u in (0, 1):
            c = c2 * 2 + u
            read_desc(c, u).wait()

            @pl.when(c >= 2)
            def _():
                for g in range(_NUM_GROUPS):
                    write_desc(c - 2, u, g).wait()

            @plsc.parallel_loop(0, _CR, step=1, unroll=2)
            def _(r):
                for g in range(_NUM_GROUPS):
                    for j in range(_NUM_SLICES):
                        src = (j * _NUM_GROUPS + g) * _CHUNK
                        dst = j * _CHUNK
                        for k in (0, 16):
                            out_bufs[u][g][r, pl.ds(dst + k, 16)] = in_bufs[
                                u
                            ][r, pl.ds(src + k, 16)]

            for g in range(_NUM_GROUPS):
                write_desc(c, u, g).start()

            @pl.when(c + 2 < _NCHUNK)
            def _():
                read_desc(c + 2, u).start()

        return 0

    lax.fori_loop(0, _NCHUNK // 2, chunk_pair, 0)

    for u in (0, 1):
        for g in range(_NUM_GROUPS):
            write_desc(_NCHUNK - 2 + u, u, g).wait()


def kernel(input_tensor):
    out_type = [
        jax.ShapeDtypeStruct((_BATCH, _GROUP_W), jnp.float32)
    ] * _NUM_GROUPS
    f = pl.kernel(
        _body,
        out_type=out_type,
        mesh=plsc.VectorSubcoreMesh(core_axis_name="c", subcore_axis_name="s"),
        scratch_types=(
            [pltpu.VMEM((_CR, _READ_W), jnp.float32)] * 2
            + [pltpu.VMEM((_CR, _GROUP_W), jnp.float32)] * (2 * _NUM_GROUPS)
            + [pltpu.SemaphoreType.DMA] * 4
        ),
        compiler_params=pltpu.CompilerParams(use_tc_tiling_on_sc=True),
    )
    return tuple(f(input_tensor))
